# Initial kernel scaffold; baseline (speedup 1.0000x reference)
import jax
import jax.numpy as jnp
from jax.experimental import pallas as pl


def kernel(x, edge_index, W_gat, att_src, att_dst, b_gat, W1, b1, W2, b2, g1, be1, g2, be2):
    raise NotImplementedError("write your pallas kernel here")


# speedup vs baseline: 16.2490x; 16.2490x over previous
import functools

import jax
import jax.numpy as jnp
from jax import lax
from jax.experimental import pallas as pl
from jax.experimental.pallas import tpu as pltpu
from jax.experimental.pallas import tpu_sc as plsc

N = 10000
D = 128
H = 8
C = 128
FF = 128

NC = 2
NS = 16
NW = NC * NS

NP = 10240
K1 = 128
K3 = 32
EALIGN = NW * K1

ROWS_PER_TILE = NP // NS


def _leaky_exp(t):
    return jnp.exp(jnp.maximum(t, 0.2 * t))


def _p0_body(x_ref, wg_ref, acat_ref, h_ref, acomb_ref):
    hb = jnp.dot(x_ref[...], wg_ref[...], preferred_element_type=jnp.float32)
    h_ref[...] = hb
    acomb_ref[...] = jnp.dot(hb, acat_ref[...],
                             preferred_element_type=jnp.float32)


def _p0(x_pad, w_gat, acat):
    bp = 512
    grid = (NP // bp,)
    return pl.pallas_call(
        _p0_body,
        grid=grid,
        in_specs=[
            pl.BlockSpec((bp, D), lambda i: (i, 0)),
            pl.BlockSpec((D, H * C), lambda i: (0, 0)),
            pl.BlockSpec((H * C, 128), lambda i: (0, 0)),
        ],
        out_specs=[
            pl.BlockSpec((bp, H * C), lambda i: (i, 0)),
            pl.BlockSpec((bp, 128), lambda i: (i, 0)),
        ],
        out_shape=[
            jax.ShapeDtypeStruct((NP, H * C), jnp.float32),
            jax.ShapeDtypeStruct((NP, 128), jnp.float32),
        ],
    )(x_pad, w_gat, acat)


def _sc_mesh():
    return plsc.VectorSubcoreMesh(
        core_axis_name="c", subcore_axis_name="s", num_cores=NC, num_subcores=NS
    )


_SC_PARAMS = pltpu.CompilerParams(use_tc_tiling_on_sc=False)


def _zero_rows(ref, ncols16):
    z = jnp.zeros((16,), jnp.float32)

    def body(r, _):
        for j in range(ncols16):
            ref[r, pl.ds(j * 16, 16)] = z
        return 0

    lax.fori_loop(0, ref.shape[0], body, 0)


def _p1_body(ep, src_hbm, dst_hbm, acomb_hbm, w_hbm, asum_hbm,
             idx_s, idx_d, srow, drow, wrow, stage, acc_sh, sem):
    c = lax.axis_index("c")
    s = lax.axis_index("s")
    wid = s * NC + c

    _zero_rows(stage, 1)
    r0 = s * ROWS_PER_TILE
    for j in range(ROWS_PER_TILE // K1):
        pltpu.sync_copy(stage, acc_sh.at[pl.ds(r0 + j * K1, K1)])
    plsc.subcore_barrier()

    per_w = ep // K1 // NW

    def chunk(i, _):
        base = (wid * per_w + i) * K1
        pltpu.sync_copy(src_hbm.at[pl.ds(base, K1)], idx_s)
        pltpu.sync_copy(dst_hbm.at[pl.ds(base, K1)], idx_d)
        pltpu.async_copy(acomb_hbm.at[idx_s], srow, sem).wait()
        pltpu.async_copy(acomb_hbm.at[idx_d], drow, sem).wait()

        def erow(e, _):
            wrow[e] = _leaky_exp(srow[e, pl.ds(0, 16)]
                                 + drow[e, pl.ds(16, 16)])
            return 0

        lax.fori_loop(0, K1, erow, 0)
        pltpu.sync_copy(wrow, w_hbm.at[pl.ds(base, K1)])
        pltpu.sync_copy(wrow, acc_sh.at[idx_d], add=True)
        return 0

    lax.fori_loop(0, per_w, chunk, 0)
    plsc.subcore_barrier()

    for j in range(ROWS_PER_TILE // K1):
        pltpu.sync_copy(acc_sh.at[pl.ds(r0 + j * K1, K1)], stage)
        pltpu.sync_copy(stage, asum_hbm.at[c, pl.ds(r0 + j * K1, K1)])


def _p1(src, dst, acomb, ep):
    kfn = functools.partial(
        pl.kernel,
        out_type=[
            jax.ShapeDtypeStruct((ep, 16), jnp.float32),
            jax.ShapeDtypeStruct((NC, NP, 16), jnp.float32),
        ],
        mesh=_sc_mesh(),
        compiler_params=_SC_PARAMS,
        scratch_types=[
            pltpu.VMEM((K1,), jnp.int32),
            pltpu.VMEM((K1,), jnp.int32),
            pltpu.VMEM((K1, 128), jnp.float32),
            pltpu.VMEM((K1, 128), jnp.float32),
            pltpu.VMEM((K1, 16), jnp.float32),
            pltpu.VMEM((K1, 16), jnp.float32),
            pltpu.VMEM_SHARED((NP, 16), jnp.float32),
            pltpu.SemaphoreType.DMA,
        ],
    )(functools.partial(_p1_body, ep))
    return kfn(src, dst, acomb)


def _p2_body(asum_ref, rec_ref):
    a = asum_ref[0] + asum_ref[1]
    r = (1.0 / H) / (a + 1e-16)
    bp = r.shape[0]
    rec_ref[...] = jnp.concatenate(
        [r, jnp.zeros((bp, 112), jnp.float32)], axis=-1)


def _p2(asum):
    bp = 1024
    return pl.pallas_call(
        _p2_body,
        grid=(NP // bp,),
        in_specs=[pl.BlockSpec((NC, bp, 16), lambda i: (0, i, 0))],
        out_specs=pl.BlockSpec((bp, 128), lambda i: (i, 0)),
        out_shape=jax.ShapeDtypeStruct((NP, 128), jnp.float32),
    )(asum)


def _p3_body(ep, src_hbm, dst_hbm, w_hbm, rec_hbm, h_hbm, out_hbm,
             idx_s, idx_d, wrow, recrow, hrows, mbuf, acc_sh, sem):
    c = lax.axis_index("c")
    s = lax.axis_index("s")
    wid = s * NC + c

    _zero_rows(mbuf, C // 16)
    r0 = s * ROWS_PER_TILE
    for j in range(ROWS_PER_TILE // K3):
        pltpu.sync_copy(mbuf, acc_sh.at[pl.ds(r0 + j * K3, K3)])
    plsc.subcore_barrier()

    per_w = ep // K3 // NW

    def chunk(i, _):
        base = (wid * per_w + i) * K3
        pltpu.sync_copy(src_hbm.at[pl.ds(base, K3)], idx_s)
        pltpu.sync_copy(dst_hbm.at[pl.ds(base, K3)], idx_d)
        pltpu.sync_copy(w_hbm.at[pl.ds(base, K3)], wrow)
        pltpu.async_copy(rec_hbm.at[idx_d], recrow, sem).wait()
        pltpu.async_copy(h_hbm.at[idx_s], hrows, sem).wait()

        def emsg(e, _):
            cvec = wrow[e] * recrow[e, pl.ds(0, 16)]
            acc = [None] * (C // 16)
            for h in range(H):
                sp = jnp.full((16,), cvec[h])
                for j in range(C // 16):
                    v = sp * hrows[e, pl.ds(h * C + j * 16, 16)]
                    acc[j] = v if h == 0 else acc[j] + v
            for j in range(C // 16):
                mbuf[e, pl.ds(j * 16, 16)] = acc[j]
            return 0

        lax.fori_loop(0, K3, emsg, 0)
        pltpu.sync_copy(mbuf, acc_sh.at[idx_d], add=True)
        return 0

    lax.fori_loop(0, per_w, chunk, 0)
    plsc.subcore_barrier()

    for j in range(ROWS_PER_TILE // K3):
        pltpu.sync_copy(acc_sh.at[pl.ds(r0 + j * K3, K3)], mbuf)
        pltpu.sync_copy(mbuf, out_hbm.at[c, pl.ds(r0 + j * K3, K3)])


def _p3(src, dst, w, rec, h, ep):
    kfn = functools.partial(
        pl.kernel,
        out_type=jax.ShapeDtypeStruct((NC, NP, C), jnp.float32),
        mesh=_sc_mesh(),
        compiler_params=_SC_PARAMS,
        scratch_types=[
            pltpu.VMEM((K3,), jnp.int32),
            pltpu.VMEM((K3,), jnp.int32),
            pltpu.VMEM((K3, 16), jnp.float32),
            pltpu.VMEM((K3, 128), jnp.float32),
            pltpu.VMEM((K3, H * C), jnp.float32),
            pltpu.VMEM((K3, C), jnp.float32),
            pltpu.VMEM_SHARED((NP, C), jnp.float32),
            pltpu.SemaphoreType.DMA,
        ],
    )(functools.partial(_p3_body, ep))
    return kfn(src, dst, w, rec, h)


def _ln(x, g, b):
    mu = jnp.mean(x, axis=-1, keepdims=True)
    xc = x - mu
    var = jnp.mean(xc * xc, axis=-1, keepdims=True)
    return xc * jax.lax.rsqrt(var + 1e-5) * g + b


def _p4_body(x_ref, op_ref, bg_ref, w1_ref, b1_ref, w2_ref, b2_ref,
             g1_ref, be1_ref, g2_ref, be2_ref, y_ref):
    out = op_ref[0] + op_ref[1] + bg_ref[...]
    x1 = _ln(out + x_ref[...], g1_ref[...], be1_ref[...])
    hmid = jnp.maximum(
        jnp.dot(x1, w1_ref[...], preferred_element_type=jnp.float32)
        + b1_ref[...], 0.0)
    f = jnp.dot(hmid, w2_ref[...], preferred_element_type=jnp.float32) \
        + b2_ref[...]
    y_ref[...] = _ln(f + x1, g2_ref[...], be2_ref[...])


def _p4(x, outp, b_gat, w1, b1, w2, b2, g1, be1, g2, be2):
    bp = 1000
    vec = lambda i: (0,)
    return pl.pallas_call(
        _p4_body,
        grid=(N // bp,),
        in_specs=[
            pl.BlockSpec((bp, C), lambda i: (i, 0)),
            pl.BlockSpec((NC, bp, C), lambda i: (0, i, 0)),
            pl.BlockSpec((C,), vec),
            pl.BlockSpec((C, FF), lambda i: (0, 0)),
            pl.BlockSpec((FF,), vec),
            pl.BlockSpec((FF, C), lambda i: (0, 0)),
            pl.BlockSpec((C,), vec),
            pl.BlockSpec((C,), vec),
            pl.BlockSpec((C,), vec),
            pl.BlockSpec((C,), vec),
            pl.BlockSpec((C,), vec),
        ],
        out_specs=pl.BlockSpec((bp, C), lambda i: (i, 0)),
        out_shape=jax.ShapeDtypeStruct((N, C), jnp.float32),
    )(x, outp, b_gat, w1, b1, w2, b2, g1, be1, g2, be2)


def kernel(x, edge_index, W_gat, att_src, att_dst, b_gat,
           W1, b1, W2, b2, g1, be1, g2, be2):
    loop = jnp.arange(N, dtype=jnp.int32)
    src = jnp.concatenate([edge_index[0].astype(jnp.int32), loop])
    dst = jnp.concatenate([edge_index[1].astype(jnp.int32), loop])
    e2 = src.shape[0]
    ep = -(-e2 // EALIGN) * EALIGN
    pad = ep - e2
    src = jnp.concatenate([src, jnp.full((pad,), N, jnp.int32)])
    dst = jnp.concatenate([dst, jnp.full((pad,), N, jnp.int32)])

    x_pad = jnp.pad(x, ((0, NP - N), (0, 0)))

    eye = jnp.eye(H, dtype=jnp.float32)
    a_s = (att_src[0][:, :, None] * eye[:, None, :]).reshape(H * C, H)
    a_d = (att_dst[0][:, :, None] * eye[:, None, :]).reshape(H * C, H)
    acat = jnp.concatenate(
        [a_s, a_s, a_d, a_d, jnp.zeros((H * C, 96), jnp.float32)], axis=1)

    h, acomb = _p0(x_pad, W_gat, acat)
    w, asum = _p1(src, dst, acomb, ep)
    rec = _p2(asum)
    outp = _p3(src, dst, w, rec, h, ep)
    return _p4(x, outp, b_gat, W1, b1, W2, b2, g1, be1, g2, be2)


# speedup vs baseline: 23.1964x; 1.4276x over previous
import functools

import jax
import jax.numpy as jnp
from jax import lax
from jax.experimental import pallas as pl
from jax.experimental.pallas import tpu as pltpu
from jax.experimental.pallas import tpu_sc as plsc

N = 10000
D = 128
H = 8
C = 128
FF = 128

NC = 2
NS = 16
NW = NC * NS

NP = 10240
K1 = 128
K3 = 16
EALIGN = 2 * NW * K1

ROWS_PER_TILE = NP // NS


def _leaky_exp(t):
    return jnp.exp(jnp.maximum(t, 0.2 * t))


def _p0_body(x_ref, wg_ref, acat_ref, h_ref, acomb_ref):
    hb = jnp.dot(x_ref[...], wg_ref[...], preferred_element_type=jnp.float32)
    h_ref[...] = hb.astype(jnp.bfloat16)
    acomb_ref[...] = jnp.dot(hb, acat_ref[...],
                             preferred_element_type=jnp.float32)


def _p0(x_pad, w_gat, acat):
    bp = 512
    grid = (NP // bp,)
    return pl.pallas_call(
        _p0_body,
        grid=grid,
        in_specs=[
            pl.BlockSpec((bp, D), lambda i: (i, 0)),
            pl.BlockSpec((D, H * C), lambda i: (0, 0)),
            pl.BlockSpec((H * C, 128), lambda i: (0, 0)),
        ],
        out_specs=[
            pl.BlockSpec((bp, H * C), lambda i: (i, 0)),
            pl.BlockSpec((bp, 128), lambda i: (i, 0)),
        ],
        out_shape=[
            jax.ShapeDtypeStruct((NP, H * C), jnp.bfloat16),
            jax.ShapeDtypeStruct((NP, 128), jnp.float32),
        ],
    )(x_pad, w_gat, acat)


def _sc_mesh():
    return plsc.VectorSubcoreMesh(
        core_axis_name="c", subcore_axis_name="s", num_cores=NC, num_subcores=NS
    )


_SC_PARAMS = pltpu.CompilerParams(use_tc_tiling_on_sc=False,
                                  needs_layout_passes=False)


def _zero_rows(ref, ncols16):
    z = jnp.zeros((16,), jnp.float32)

    def body(r, _):
        for j in range(ncols16):
            ref[r, pl.ds(j * 16, 16)] = z
        return 0

    lax.fori_loop(0, ref.shape[0], body, 0)


def _p1_body(ep, src_hbm, dst_hbm, acomb_hbm, w_hbm, asum_hbm,
             idx_s0, idx_d0, idx_s1, idx_d1,
             srow0, drow0, srow1, drow1,
             wrow, stage, acc_sh, semg0, semg1, semi0, semi1):
    c = lax.axis_index("c")
    s = lax.axis_index("s")
    wid = s * NC + c
    idx_s = (idx_s0, idx_s1)
    idx_d = (idx_d0, idx_d1)
    srow = (srow0, srow1)
    drow = (drow0, drow1)
    semg = (semg0, semg1)
    semi = (semi0, semi1)

    _zero_rows(stage, 1)
    r0 = s * ROWS_PER_TILE
    for j in range(ROWS_PER_TILE // K1):
        pltpu.sync_copy(stage, acc_sh.at[pl.ds(r0 + j * K1, K1)])
    plsc.subcore_barrier()

    per_w = ep // K1 // NW
    base0 = wid * per_w
    last = per_w - 1

    def idx_issue(jj, b):
        bs = (base0 + jj) * K1
        pltpu.async_copy(src_hbm.at[pl.ds(bs, K1)], idx_s[b], semi[b])
        pltpu.async_copy(dst_hbm.at[pl.ds(bs, K1)], idx_d[b], semi[b])

    def idx_wait(b):
        pltpu.make_async_copy(
            src_hbm.at[pl.ds(0, K1)], idx_s[b], semi[b]).wait()
        pltpu.make_async_copy(
            dst_hbm.at[pl.ds(0, K1)], idx_d[b], semi[b]).wait()

    def g_issue(b):
        pltpu.async_copy(acomb_hbm.at[idx_s[b]], srow[b], semg[b])
        pltpu.async_copy(acomb_hbm.at[idx_d[b]], drow[b], semg[b])

    def g_wait(b):
        pltpu.make_async_copy(acomb_hbm.at[idx_s[b]], srow[b], semg[b]).wait()
        pltpu.make_async_copy(acomb_hbm.at[idx_d[b]], drow[b], semg[b]).wait()

    pltpu.sync_copy(src_hbm.at[pl.ds(base0 * K1, K1)], idx_s0)
    pltpu.sync_copy(dst_hbm.at[pl.ds(base0 * K1, K1)], idx_d0)
    g_issue(0)
    idx_issue(jnp.minimum(1, last), 1)

    def pair(i, _):
        for b in (0, 1):
            j = 2 * i + b
            nb = 1 - b
            idx_wait(nb)
            g_issue(nb)
            g_wait(b)

            def erow(e, _):
                wrow[e] = _leaky_exp(srow[b][e, pl.ds(0, 16)]
                                     + drow[b][e, pl.ds(16, 16)])
                return 0

            lax.fori_loop(0, K1, erow, 0)
            pltpu.sync_copy(wrow, acc_sh.at[idx_d[b]], add=True)
            pltpu.sync_copy(wrow, w_hbm.at[pl.ds((base0 + j) * K1, K1)])
            idx_issue(jnp.minimum(j + 2, last), b)
        return 0

    lax.fori_loop(0, per_w // 2, pair, 0)
    g_wait(0)
    idx_wait(1)
    plsc.subcore_barrier()

    for j in range(ROWS_PER_TILE // K1):
        pltpu.sync_copy(acc_sh.at[pl.ds(r0 + j * K1, K1)], stage)
        pltpu.sync_copy(stage, asum_hbm.at[c, pl.ds(r0 + j * K1, K1)])


def _p1(src, dst, acomb, ep):
    kfn = functools.partial(
        pl.kernel,
        out_type=[
            jax.ShapeDtypeStruct((ep, 16), jnp.float32),
            jax.ShapeDtypeStruct((NC, NP, 16), jnp.float32),
        ],
        mesh=_sc_mesh(),
        compiler_params=_SC_PARAMS,
        scratch_types=[
            pltpu.VMEM((K1,), jnp.int32),
            pltpu.VMEM((K1,), jnp.int32),
            pltpu.VMEM((K1,), jnp.int32),
            pltpu.VMEM((K1,), jnp.int32),
            pltpu.VMEM((K1, 128), jnp.float32),
            pltpu.VMEM((K1, 128), jnp.float32),
            pltpu.VMEM((K1, 128), jnp.float32),
            pltpu.VMEM((K1, 128), jnp.float32),
            pltpu.VMEM((K1, 16), jnp.float32),
            pltpu.VMEM((K1, 16), jnp.float32),
            pltpu.VMEM_SHARED((NP, 16), jnp.float32),
            pltpu.SemaphoreType.DMA,
            pltpu.SemaphoreType.DMA,
            pltpu.SemaphoreType.DMA,
            pltpu.SemaphoreType.DMA,
        ],
    )(functools.partial(_p1_body, ep))
    return kfn(src, dst, acomb)


def _p2_body(asum_ref, rec_ref):
    a = asum_ref[0] + asum_ref[1]
    r = (1.0 / H) / (a + 1e-16)
    bp = r.shape[0]
    rec_ref[...] = jnp.concatenate(
        [r, jnp.zeros((bp, 112), jnp.float32)], axis=-1)


def _p2(asum):
    bp = 1024
    return pl.pallas_call(
        _p2_body,
        grid=(NP // bp,),
        in_specs=[pl.BlockSpec((NC, bp, 16), lambda i: (0, i, 0))],
        out_specs=pl.BlockSpec((bp, 128), lambda i: (i, 0)),
        out_shape=jax.ShapeDtypeStruct((NP, 128), jnp.float32),
    )(asum)


def _p3_body(ep, src_hbm, dst_hbm, w_hbm, rec_hbm, h_hbm, z_hbm, out_hbm,
             idx_s0, idx_d0, idx_s1, idx_d1,
             wrow0, wrow1, recrow0, recrow1, hrows0, hrows1,
             mbuf, acc_sh, semg0, semg1, semi0, semi1):
    c = lax.axis_index("c")
    s = lax.axis_index("s")
    wid = s * NC + c
    idx_s = (idx_s0, idx_s1)
    idx_d = (idx_d0, idx_d1)
    wrow = (wrow0, wrow1)
    recrow = (recrow0, recrow1)
    hrows = (hrows0, hrows1)
    semg = (semg0, semg1)
    semi = (semi0, semi1)

    r0 = s * ROWS_PER_TILE
    pltpu.sync_copy(z_hbm, acc_sh.at[pl.ds(r0, ROWS_PER_TILE)])
    plsc.subcore_barrier()

    per_w = ep // K3 // NW
    base0 = wid * per_w
    last = per_w - 1

    def idx_issue(jj, b):
        bs = (base0 + jj) * K3
        pltpu.async_copy(src_hbm.at[pl.ds(bs, K3)], idx_s[b], semi[b])
        pltpu.async_copy(dst_hbm.at[pl.ds(bs, K3)], idx_d[b], semi[b])

    def idx_wait(b):
        pltpu.make_async_copy(
            src_hbm.at[pl.ds(0, K3)], idx_s[b], semi[b]).wait()
        pltpu.make_async_copy(
            dst_hbm.at[pl.ds(0, K3)], idx_d[b], semi[b]).wait()

    def g_issue(jj, b):
        bs = (base0 + jj) * K3
        pltpu.async_copy(w_hbm.at[pl.ds(bs, K3)], wrow[b], semg[b])
        pltpu.async_copy(rec_hbm.at[idx_d[b]], recrow[b], semg[b])
        pltpu.async_copy(h_hbm.at[idx_s[b]], hrows[b], semg[b])

    def g_wait(b):
        pltpu.make_async_copy(
            w_hbm.at[pl.ds(0, K3)], wrow[b], semg[b]).wait()
        pltpu.make_async_copy(rec_hbm.at[idx_d[b]], recrow[b], semg[b]).wait()
        pltpu.make_async_copy(h_hbm.at[idx_s[b]], hrows[b], semg[b]).wait()

    pltpu.sync_copy(src_hbm.at[pl.ds(base0 * K3, K3)], idx_s0)
    pltpu.sync_copy(dst_hbm.at[pl.ds(base0 * K3, K3)], idx_d0)
    g_issue(0, 0)
    idx_issue(jnp.minimum(1, last), 1)

    def pair(i, _):
        for b in (0, 1):
            j = 2 * i + b
            nb = 1 - b
            idx_wait(nb)
            g_issue(jnp.minimum(j + 1, last), nb)
            g_wait(b)

            def emsg(e, _):
                cvec = wrow[b][e] * recrow[b][e, pl.ds(0, 16)]
                acc_ev = [None] * (C // 32)
                acc_od = [None] * (C // 32)
                for h in range(H):
                    sp = jnp.full((16,), cvec[h])
                    for g in range(C // 32):
                        v32 = hrows[b][e, pl.ds(h * C + g * 32, 32)]
                        ev, od = plsc.unpack(
                            v32, format=plsc.PackFormat.INTERLEAVED)
                        if h == 0:
                            acc_ev[g] = sp * ev
                            acc_od[g] = sp * od
                        else:
                            acc_ev[g] = acc_ev[g] + sp * ev
                            acc_od[g] = acc_od[g] + sp * od
                for g in range(C // 32):
                    mbuf[e, pl.ds(g * 32, 16)] = acc_ev[g]
                    mbuf[e, pl.ds(g * 32 + 16, 16)] = acc_od[g]
                return 0

            lax.fori_loop(0, K3, emsg, 0)
            pltpu.sync_copy(mbuf, acc_sh.at[idx_d[b]], add=True)
            idx_issue(jnp.minimum(j + 2, last), b)
        return 0

    lax.fori_loop(0, per_w // 2, pair, 0)
    g_wait(0)
    idx_wait(1)
    plsc.subcore_barrier()

    pltpu.sync_copy(acc_sh.at[pl.ds(r0, ROWS_PER_TILE)],
                    out_hbm.at[c, pl.ds(r0, ROWS_PER_TILE)])


def _p3(src, dst, w, rec, h, zeros, ep):
    kfn = functools.partial(
        pl.kernel,
        out_type=jax.ShapeDtypeStruct((NC, NP, C), jnp.float32),
        mesh=_sc_mesh(),
        compiler_params=_SC_PARAMS,
        scratch_types=[
            pltpu.VMEM((K3,), jnp.int32),
            pltpu.VMEM((K3,), jnp.int32),
            pltpu.VMEM((K3,), jnp.int32),
            pltpu.VMEM((K3,), jnp.int32),
            pltpu.VMEM((K3, 16), jnp.float32),
            pltpu.VMEM((K3, 16), jnp.float32),
            pltpu.VMEM((K3, 128), jnp.float32),
            pltpu.VMEM((K3, 128), jnp.float32),
            pltpu.VMEM((K3, H * C), jnp.bfloat16),
            pltpu.VMEM((K3, H * C), jnp.bfloat16),
            pltpu.VMEM((K3, C), jnp.float32),
            pltpu.VMEM_SHARED((NP, C), jnp.float32),
            pltpu.SemaphoreType.DMA,
            pltpu.SemaphoreType.DMA,
            pltpu.SemaphoreType.DMA,
            pltpu.SemaphoreType.DMA,
        ],
    )(functools.partial(_p3_body, ep))
    return kfn(src, dst, w, rec, h, zeros)


def _ln(x, g, b):
    mu = jnp.mean(x, axis=-1, keepdims=True)
    xc = x - mu
    var = jnp.mean(xc * xc, axis=-1, keepdims=True)
    return xc * jax.lax.rsqrt(var + 1e-5) * g + b


def _p4_body(x_ref, op_ref, pm_ref, bg_ref, w1_ref, b1_ref, w2_ref, b2_ref,
             g1_ref, be1_ref, g2_ref, be2_ref, y_ref):
    out = jnp.dot(op_ref[0] + op_ref[1], pm_ref[...],
                  preferred_element_type=jnp.float32) + bg_ref[...]
    x1 = _ln(out + x_ref[...], g1_ref[...], be1_ref[...])
    hmid = jnp.maximum(
        jnp.dot(x1, w1_ref[...], preferred_element_type=jnp.float32)
        + b1_ref[...], 0.0)
    f = jnp.dot(hmid, w2_ref[...], preferred_element_type=jnp.float32) \
        + b2_ref[...]
    y_ref[...] = _ln(f + x1, g2_ref[...], be2_ref[...])


def _p4(x, outp, pmat, b_gat, w1, b1, w2, b2, g1, be1, g2, be2):
    bp = 1000
    vec = lambda i: (0,)
    return pl.pallas_call(
        _p4_body,
        grid=(N // bp,),
        in_specs=[
            pl.BlockSpec((bp, C), lambda i: (i, 0)),
            pl.BlockSpec((NC, bp, C), lambda i: (0, i, 0)),
            pl.BlockSpec((C, C), lambda i: (0, 0)),
            pl.BlockSpec((C,), vec),
            pl.BlockSpec((C, FF), lambda i: (0, 0)),
            pl.BlockSpec((FF,), vec),
            pl.BlockSpec((FF, C), lambda i: (0, 0)),
            pl.BlockSpec((C,), vec),
            pl.BlockSpec((C,), vec),
            pl.BlockSpec((C,), vec),
            pl.BlockSpec((C,), vec),
            pl.BlockSpec((C,), vec),
        ],
        out_specs=pl.BlockSpec((bp, C), lambda i: (i, 0)),
        out_shape=jax.ShapeDtypeStruct((N, C), jnp.float32),
    )(x, outp, pmat, b_gat, w1, b1, w2, b2, g1, be1, g2, be2)


def kernel(x, edge_index, W_gat, att_src, att_dst, b_gat,
           W1, b1, W2, b2, g1, be1, g2, be2):
    loop = jnp.arange(N, dtype=jnp.int32)
    src = jnp.concatenate([edge_index[0].astype(jnp.int32), loop])
    dst = jnp.concatenate([edge_index[1].astype(jnp.int32), loop])
    e2 = src.shape[0]
    ep = -(-e2 // EALIGN) * EALIGN
    pad = ep - e2
    src = jnp.concatenate([src, jnp.full((pad,), N, jnp.int32)])
    dst = jnp.concatenate([dst, jnp.full((pad,), N, jnp.int32)])

    x_pad = jnp.pad(x, ((0, NP - N), (0, 0)))

    eye = jnp.eye(H, dtype=jnp.float32)
    a_s = (att_src[0][:, :, None] * eye[:, None, :]).reshape(H * C, H)
    a_d = (att_dst[0][:, :, None] * eye[:, None, :]).reshape(H * C, H)
    acat = jnp.concatenate(
        [a_s, a_s, a_d, a_d, jnp.zeros((H * C, 96), jnp.float32)], axis=1)

    col = jnp.arange(C)
    g32, r32 = col // 32, col % 32
    pidx = jnp.where(r32 % 2 == 0, g32 * 32 + r32 // 2,
                     g32 * 32 + 16 + r32 // 2)
    pmat = jnp.zeros((C, C), jnp.float32).at[pidx, col].set(1.0)

    zeros = jnp.zeros((ROWS_PER_TILE, C), jnp.float32)

    h, acomb = _p0(x_pad, W_gat, acat)
    w, asum = _p1(src, dst, acomb, ep)
    rec = _p2(asum)
    outp = _p3(src, dst, w, rec, h, zeros, ep)
    return _p4(x, outp, pmat, b_gat, W1, b1, W2, b2, g1, be1, g2, be2)


# speedup vs baseline: 24.1242x; 1.0400x over previous
import functools

import jax
import jax.numpy as jnp
from jax import lax
from jax.experimental import pallas as pl
from jax.experimental.pallas import tpu as pltpu
from jax.experimental.pallas import tpu_sc as plsc

N = 10000
D = 128
H = 8
C = 128
FF = 128

NC = 2
NS = 16
NW = NC * NS

NP = 10240
K1 = 128
K3 = 16
EALIGN = 2 * NW * K1

ROWS_PER_TILE = NP // NS


def _leaky_exp(t):
    return jnp.exp(jnp.maximum(t, 0.2 * t))


def _p0_body(x_ref, wg_ref, acat_ref, h_ref, acomb_ref):
    hb = jnp.dot(x_ref[...], wg_ref[...], preferred_element_type=jnp.float32)
    h_ref[...] = hb.astype(jnp.bfloat16)
    acomb_ref[...] = jnp.dot(hb, acat_ref[...],
                             preferred_element_type=jnp.float32)


def _p0(x_pad, w_gat, acat):
    bp = 512
    grid = (NP // bp,)
    return pl.pallas_call(
        _p0_body,
        grid=grid,
        in_specs=[
            pl.BlockSpec((bp, D), lambda i: (i, 0)),
            pl.BlockSpec((D, H * C), lambda i: (0, 0)),
            pl.BlockSpec((H * C, 128), lambda i: (0, 0)),
        ],
        out_specs=[
            pl.BlockSpec((bp, H * C), lambda i: (i, 0)),
            pl.BlockSpec((bp, 128), lambda i: (i, 0)),
        ],
        out_shape=[
            jax.ShapeDtypeStruct((NP, H * C), jnp.bfloat16),
            jax.ShapeDtypeStruct((NP, 128), jnp.float32),
        ],
    )(x_pad, w_gat, acat)


def _sc_mesh():
    return plsc.VectorSubcoreMesh(
        core_axis_name="c", subcore_axis_name="s", num_cores=NC, num_subcores=NS
    )


_SC_PARAMS = pltpu.CompilerParams(use_tc_tiling_on_sc=False,
                                  needs_layout_passes=False)


def _zero_rows(ref, ncols16):
    z = jnp.zeros((16,), jnp.float32)

    def body(r, _):
        for j in range(ncols16):
            ref[r, pl.ds(j * 16, 16)] = z
        return 0

    lax.fori_loop(0, ref.shape[0], body, 0)


def _p1_body(ep, src_hbm, dst_hbm, acomb_hbm, w_hbm, asum_hbm,
             idx_s0, idx_d0, idx_s1, idx_d1, idxx0, idxx1,
             srow0, drow0, srow1, drow1, wrow0, wrow1,
             stage, acc_sh, semg0, semg1, semi0, semi1,
             sems0, sems1, semw0, semw1):
    c = lax.axis_index("c")
    s = lax.axis_index("s")
    wid = s * NC + c
    idx_s = (idx_s0, idx_s1)
    idx_d = (idx_d0, idx_d1)
    idxx = (idxx0, idxx1)
    srow = (srow0, srow1)
    drow = (drow0, drow1)
    wrow = (wrow0, wrow1)
    semg = (semg0, semg1)
    semi = (semi0, semi1)
    sems = (sems0, sems1)
    semw = (semw0, semw1)

    _zero_rows(stage, 1)
    r0 = s * ROWS_PER_TILE
    for j in range(ROWS_PER_TILE // K1):
        pltpu.sync_copy(stage, acc_sh.at[pl.ds(r0 + j * K1, K1)])
    plsc.subcore_barrier()

    per_w = ep // K1 // NW
    base0 = wid * per_w
    last = per_w - 1

    def idx_issue(jj, b):
        bs = (base0 + jj) * K1
        pltpu.async_copy(src_hbm.at[pl.ds(bs, K1)], idx_s[b], semi[b])
        pltpu.async_copy(dst_hbm.at[pl.ds(bs, K1)], idx_d[b], semi[b])

    def idx_wait(b):
        pltpu.make_async_copy(
            src_hbm.at[pl.ds(0, K1)], idx_s[b], semi[b]).wait()
        pltpu.make_async_copy(
            dst_hbm.at[pl.ds(0, K1)], idx_d[b], semi[b]).wait()

    def g_issue(b):
        pltpu.async_copy(acomb_hbm.at[idx_s[b]], srow[b], semg[b])
        pltpu.async_copy(acomb_hbm.at[idx_d[b]], drow[b], semg[b])

    def g_wait(b):
        pltpu.make_async_copy(acomb_hbm.at[idx_s[b]], srow[b], semg[b]).wait()
        pltpu.make_async_copy(acomb_hbm.at[idx_d[b]], drow[b], semg[b]).wait()

    def s_wait(b):
        pltpu.make_async_copy(wrow[b], acc_sh.at[idxx[b]], sems[b]).wait()

    def w_wait(b):
        pltpu.make_async_copy(
            wrow[b], w_hbm.at[pl.ds(0, K1)], semw[b]).wait()

    pltpu.sync_copy(src_hbm.at[pl.ds(base0 * K1, K1)], idx_s0)
    pltpu.sync_copy(dst_hbm.at[pl.ds(base0 * K1, K1)], idx_d0)
    g_issue(0)
    idx_issue(jnp.minimum(1, last), 1)
    _zero_rows(wrow0, 1)
    _zero_rows(wrow1, 1)
    zi = jnp.zeros((16,), jnp.int32)
    for t in range(K1 // 16):
        idxx1[pl.ds(t * 16, 16)] = zi
    pltpu.async_copy(wrow1, acc_sh.at[idxx1], sems1, add=True)
    pltpu.async_copy(wrow0, w_hbm.at[pl.ds(base0 * K1, K1)], semw0)
    pltpu.async_copy(
        wrow1, w_hbm.at[pl.ds((base0 + jnp.minimum(1, last)) * K1, K1)],
        semw1)

    def pair(i, _):
        for b in (0, 1):
            j = 2 * i + b
            nb = 1 - b
            idx_wait(nb)
            g_issue(nb)
            g_wait(b)
            w_wait(b)

            def erow(e, _):
                wrow[b][e] = _leaky_exp(srow[b][e, pl.ds(0, 16)]
                                        + drow[b][e, pl.ds(16, 16)])
                return 0

            lax.fori_loop(0, K1, erow, 0)
            s_wait(nb)
            for t in range(K1 // 16):
                idxx[b][pl.ds(t * 16, 16)] = idx_d[b][pl.ds(t * 16, 16)]
            pltpu.async_copy(wrow[b], acc_sh.at[idxx[b]], sems[b], add=True)
            pltpu.async_copy(
                wrow[b], w_hbm.at[pl.ds((base0 + j) * K1, K1)], semw[b])
            idx_issue(jnp.minimum(j + 2, last), b)
        return 0

    lax.fori_loop(0, per_w // 2, pair, 0)
    g_wait(0)
    idx_wait(1)
    s_wait(1)
    w_wait(0)
    w_wait(1)
    plsc.subcore_barrier()

    for j in range(ROWS_PER_TILE // K1):
        pltpu.sync_copy(acc_sh.at[pl.ds(r0 + j * K1, K1)], stage)
        pltpu.sync_copy(stage, asum_hbm.at[c, pl.ds(r0 + j * K1, K1)])


def _p1(src, dst, acomb, ep):
    kfn = functools.partial(
        pl.kernel,
        out_type=[
            jax.ShapeDtypeStruct((ep, 16), jnp.float32),
            jax.ShapeDtypeStruct((NC, NP, 16), jnp.float32),
        ],
        mesh=_sc_mesh(),
        compiler_params=_SC_PARAMS,
        scratch_types=[
            pltpu.VMEM((K1,), jnp.int32),
            pltpu.VMEM((K1,), jnp.int32),
            pltpu.VMEM((K1,), jnp.int32),
            pltpu.VMEM((K1,), jnp.int32),
            pltpu.VMEM((K1,), jnp.int32),
            pltpu.VMEM((K1,), jnp.int32),
            pltpu.VMEM((K1, 128), jnp.float32),
            pltpu.VMEM((K1, 128), jnp.float32),
            pltpu.VMEM((K1, 128), jnp.float32),
            pltpu.VMEM((K1, 128), jnp.float32),
            pltpu.VMEM((K1, 16), jnp.float32),
            pltpu.VMEM((K1, 16), jnp.float32),
            pltpu.VMEM((K1, 16), jnp.float32),
            pltpu.VMEM_SHARED((NP, 16), jnp.float32),
            pltpu.SemaphoreType.DMA,
            pltpu.SemaphoreType.DMA,
            pltpu.SemaphoreType.DMA,
            pltpu.SemaphoreType.DMA,
            pltpu.SemaphoreType.DMA,
            pltpu.SemaphoreType.DMA,
            pltpu.SemaphoreType.DMA,
            pltpu.SemaphoreType.DMA,
        ],
    )(functools.partial(_p1_body, ep))
    return kfn(src, dst, acomb)


def _p2_body(asum_ref, rec_ref):
    a = asum_ref[0] + asum_ref[1]
    r = (1.0 / H) / (a + 1e-16)
    bp = r.shape[0]
    rec_ref[...] = jnp.concatenate(
        [r, jnp.zeros((bp, 112), jnp.float32)], axis=-1)


def _p2(asum):
    bp = 1024
    return pl.pallas_call(
        _p2_body,
        grid=(NP // bp,),
        in_specs=[pl.BlockSpec((NC, bp, 16), lambda i: (0, i, 0))],
        out_specs=pl.BlockSpec((bp, 128), lambda i: (i, 0)),
        out_shape=jax.ShapeDtypeStruct((NP, 128), jnp.float32),
    )(asum)


def _p3_body(ep, src_hbm, dst_hbm, w_hbm, rec_hbm, h_hbm, z_hbm, out_hbm,
             idx_s0, idx_d0, idx_s1, idx_d1, idxx0, idxx1,
             wrow0, wrow1, recrow0, recrow1, hrows0, hrows1,
             mbuf0, mbuf1, acc_sh, semg0, semg1, semi0, semi1,
             sems0, sems1):
    c = lax.axis_index("c")
    s = lax.axis_index("s")
    wid = s * NC + c
    idx_s = (idx_s0, idx_s1)
    idx_d = (idx_d0, idx_d1)
    idxx = (idxx0, idxx1)
    wrow = (wrow0, wrow1)
    recrow = (recrow0, recrow1)
    hrows = (hrows0, hrows1)
    mbuf = (mbuf0, mbuf1)
    semg = (semg0, semg1)
    semi = (semi0, semi1)
    sems = (sems0, sems1)

    r0 = s * ROWS_PER_TILE
    pltpu.sync_copy(z_hbm, acc_sh.at[pl.ds(r0, ROWS_PER_TILE)])
    plsc.subcore_barrier()

    per_w = ep // K3 // NW
    base0 = wid * per_w
    last = per_w - 1

    def idx_issue(jj, b):
        bs = (base0 + jj) * K3
        pltpu.async_copy(src_hbm.at[pl.ds(bs, K3)], idx_s[b], semi[b])
        pltpu.async_copy(dst_hbm.at[pl.ds(bs, K3)], idx_d[b], semi[b])

    def idx_wait(b):
        pltpu.make_async_copy(
            src_hbm.at[pl.ds(0, K3)], idx_s[b], semi[b]).wait()
        pltpu.make_async_copy(
            dst_hbm.at[pl.ds(0, K3)], idx_d[b], semi[b]).wait()

    def g_issue(jj, b):
        bs = (base0 + jj) * K3
        pltpu.async_copy(w_hbm.at[pl.ds(bs, K3)], wrow[b], semg[b])
        pltpu.async_copy(rec_hbm.at[idx_d[b]], recrow[b], semg[b])
        pltpu.async_copy(h_hbm.at[idx_s[b]], hrows[b], semg[b])

    def g_wait(b):
        pltpu.make_async_copy(
            w_hbm.at[pl.ds(0, K3)], wrow[b], semg[b]).wait()
        pltpu.make_async_copy(rec_hbm.at[idx_d[b]], recrow[b], semg[b]).wait()
        pltpu.make_async_copy(h_hbm.at[idx_s[b]], hrows[b], semg[b]).wait()

    def s_wait(b):
        pltpu.make_async_copy(mbuf[b], acc_sh.at[idxx[b]], sems[b]).wait()

    pltpu.sync_copy(src_hbm.at[pl.ds(base0 * K3, K3)], idx_s0)
    pltpu.sync_copy(dst_hbm.at[pl.ds(base0 * K3, K3)], idx_d0)
    g_issue(0, 0)
    idx_issue(jnp.minimum(1, last), 1)
    _zero_rows(mbuf1, C // 16)
    idxx1[...] = jnp.zeros((16,), jnp.int32)
    pltpu.async_copy(mbuf1, acc_sh.at[idxx1], sems1, add=True)

    def pair(i, _):
        for b in (0, 1):
            j = 2 * i + b
            nb = 1 - b
            idx_wait(nb)
            g_issue(jnp.minimum(j + 1, last), nb)
            g_wait(b)
            s_wait(nb)

            def emsg(e, _):
                cvec = wrow[b][e] * recrow[b][e, pl.ds(0, 16)]
                acc_ev = [None] * (C // 32)
                acc_od = [None] * (C // 32)
                for h in range(H):
                    sp = jnp.full((16,), cvec[h])
                    for g in range(C // 32):
                        v32 = hrows[b][e, pl.ds(h * C + g * 32, 32)]
                        ev, od = plsc.unpack(
                            v32, format=plsc.PackFormat.INTERLEAVED)
                        if h == 0:
                            acc_ev[g] = sp * ev
                            acc_od[g] = sp * od
                        else:
                            acc_ev[g] = acc_ev[g] + sp * ev
                            acc_od[g] = acc_od[g] + sp * od
                for g in range(C // 32):
                    mbuf[b][e, pl.ds(g * 32, 16)] = acc_ev[g]
                    mbuf[b][e, pl.ds(g * 32 + 16, 16)] = acc_od[g]
                return 0

            lax.fori_loop(0, K3, emsg, 0)
            idxx[b][...] = idx_d[b][...]
            pltpu.async_copy(mbuf[b], acc_sh.at[idxx[b]], sems[b], add=True)
            idx_issue(jnp.minimum(j + 2, last), b)
        return 0

    lax.fori_loop(0, per_w // 2, pair, 0)
    g_wait(0)
    idx_wait(1)
    s_wait(1)
    plsc.subcore_barrier()

    pltpu.sync_copy(acc_sh.at[pl.ds(r0, ROWS_PER_TILE)],
                    out_hbm.at[c, pl.ds(r0, ROWS_PER_TILE)])


def _p3(src, dst, w, rec, h, zeros, ep):
    kfn = functools.partial(
        pl.kernel,
        out_type=jax.ShapeDtypeStruct((NC, NP, C), jnp.float32),
        mesh=_sc_mesh(),
        compiler_params=_SC_PARAMS,
        scratch_types=[
            pltpu.VMEM((K3,), jnp.int32),
            pltpu.VMEM((K3,), jnp.int32),
            pltpu.VMEM((K3,), jnp.int32),
            pltpu.VMEM((K3,), jnp.int32),
            pltpu.VMEM((K3,), jnp.int32),
            pltpu.VMEM((K3,), jnp.int32),
            pltpu.VMEM((K3, 16), jnp.float32),
            pltpu.VMEM((K3, 16), jnp.float32),
            pltpu.VMEM((K3, 128), jnp.float32),
            pltpu.VMEM((K3, 128), jnp.float32),
            pltpu.VMEM((K3, H * C), jnp.bfloat16),
            pltpu.VMEM((K3, H * C), jnp.bfloat16),
            pltpu.VMEM((K3, C), jnp.float32),
            pltpu.VMEM((K3, C), jnp.float32),
            pltpu.VMEM_SHARED((NP, C), jnp.float32),
            pltpu.SemaphoreType.DMA,
            pltpu.SemaphoreType.DMA,
            pltpu.SemaphoreType.DMA,
            pltpu.SemaphoreType.DMA,
            pltpu.SemaphoreType.DMA,
            pltpu.SemaphoreType.DMA,
        ],
    )(functools.partial(_p3_body, ep))
    return kfn(src, dst, w, rec, h, zeros)


def _ln(x, g, b):
    mu = jnp.mean(x, axis=-1, keepdims=True)
    xc = x - mu
    var = jnp.mean(xc * xc, axis=-1, keepdims=True)
    return xc * jax.lax.rsqrt(var + 1e-5) * g + b


def _p4_body(x_ref, op_ref, pm_ref, bg_ref, w1_ref, b1_ref, w2_ref, b2_ref,
             g1_ref, be1_ref, g2_ref, be2_ref, y_ref):
    out = jnp.dot(op_ref[0] + op_ref[1], pm_ref[...],
                  preferred_element_type=jnp.float32) + bg_ref[...]
    x1 = _ln(out + x_ref[...], g1_ref[...], be1_ref[...])
    hmid = jnp.maximum(
        jnp.dot(x1, w1_ref[...], preferred_element_type=jnp.float32)
        + b1_ref[...], 0.0)
    f = jnp.dot(hmid, w2_ref[...], preferred_element_type=jnp.float32) \
        + b2_ref[...]
    y_ref[...] = _ln(f + x1, g2_ref[...], be2_ref[...])


def _p4(x, outp, pmat, b_gat, w1, b1, w2, b2, g1, be1, g2, be2):
    bp = 1000
    vec = lambda i: (0,)
    return pl.pallas_call(
        _p4_body,
        grid=(N // bp,),
        in_specs=[
            pl.BlockSpec((bp, C), lambda i: (i, 0)),
            pl.BlockSpec((NC, bp, C), lambda i: (0, i, 0)),
            pl.BlockSpec((C, C), lambda i: (0, 0)),
            pl.BlockSpec((C,), vec),
            pl.BlockSpec((C, FF), lambda i: (0, 0)),
            pl.BlockSpec((FF,), vec),
            pl.BlockSpec((FF, C), lambda i: (0, 0)),
            pl.BlockSpec((C,), vec),
            pl.BlockSpec((C,), vec),
            pl.BlockSpec((C,), vec),
            pl.BlockSpec((C,), vec),
            pl.BlockSpec((C,), vec),
        ],
        out_specs=pl.BlockSpec((bp, C), lambda i: (i, 0)),
        out_shape=jax.ShapeDtypeStruct((N, C), jnp.float32),
    )(x, outp, pmat, b_gat, w1, b1, w2, b2, g1, be1, g2, be2)


def kernel(x, edge_index, W_gat, att_src, att_dst, b_gat,
           W1, b1, W2, b2, g1, be1, g2, be2):
    loop = jnp.arange(N, dtype=jnp.int32)
    src = jnp.concatenate([edge_index[0].astype(jnp.int32), loop])
    dst = jnp.concatenate([edge_index[1].astype(jnp.int32), loop])
    e2 = src.shape[0]
    ep = -(-e2 // EALIGN) * EALIGN
    pad = ep - e2
    src = jnp.concatenate([src, jnp.full((pad,), N, jnp.int32)])
    dst = jnp.concatenate([dst, jnp.full((pad,), N, jnp.int32)])

    x_pad = jnp.pad(x, ((0, NP - N), (0, 0)))

    eye = jnp.eye(H, dtype=jnp.float32)
    a_s = (att_src[0][:, :, None] * eye[:, None, :]).reshape(H * C, H)
    a_d = (att_dst[0][:, :, None] * eye[:, None, :]).reshape(H * C, H)
    acat = jnp.concatenate(
        [a_s, a_s, a_d, a_d, jnp.zeros((H * C, 96), jnp.float32)], axis=1)

    col = jnp.arange(C)
    g32, r32 = col // 32, col % 32
    pidx = jnp.where(r32 % 2 == 0, g32 * 32 + r32 // 2,
                     g32 * 32 + 16 + r32 // 2)
    pmat = jnp.zeros((C, C), jnp.float32).at[pidx, col].set(1.0)

    zeros = jnp.zeros((ROWS_PER_TILE, C), jnp.float32)

    h, acomb = _p0(x_pad, W_gat, acat)
    w, asum = _p1(src, dst, acomb, ep)
    rec = _p2(asum)
    outp = _p3(src, dst, w, rec, h, zeros, ep)
    return _p4(x, outp, pmat, b_gat, W1, b1, W2, b2, g1, be1, g2, be2)
